# baseline (speedup 1.0000x reference)
import functools

import jax
import jax.numpy as jnp
from jax import lax
from jax.experimental import pallas as pl
from jax.experimental.pallas import tpu as pltpu
from jax.experimental.pallas import tpu_sc as plsc

T = 8192
B = 4
D = 768
NC = 2
NS = 16
NW = NC * NS
ROWS_PER_W = T // NW
C = 8
LOG2C = C.bit_length() - 1
NCHUNK = ROWS_PER_W // C
NSLOT = 4
LANES = 16
G = D // LANES

_mesh = plsc.VectorSubcoreMesh(core_axis_name="c", subcore_axis_name="s")


@functools.partial(
    pl.kernel,
    mesh=_mesh,
    out_type=jax.ShapeDtypeStruct((T, B, D), jnp.float32),
    scratch_types=(
        [pltpu.VMEM((C, B, D), jnp.float32) for _ in range(NSLOT)]
        + [pltpu.VMEM((C, D), jnp.float32) for _ in range(NSLOT)]
        + [pltpu.SemaphoreType.DMA for _ in range(3 * NSLOT)]
    ),
)
def _pos_add(x_hbm, pos_hbm, out_hbm, *scratch):
    xbuf = scratch[0:NSLOT]
    pbuf = scratch[NSLOT:2 * NSLOT]
    semx = scratch[2 * NSLOT:3 * NSLOT]
    semp = scratch[3 * NSLOT:4 * NSLOT]
    semo = scratch[4 * NSLOT:5 * NSLOT]

    wid = lax.axis_index("s") * NC + lax.axis_index("c")
    row0 = wid * ROWS_PER_W

    def x_copy(ci, slot):
        r0 = row0 + ci * C
        return pltpu.make_async_copy(
            x_hbm.at[pl.ds(r0, C)], xbuf[slot], semx[slot])

    def p_copy(ci, slot):
        r0 = row0 + ci * C
        return pltpu.make_async_copy(
            pos_hbm.at[pl.ds(r0, C)], pbuf[slot], semp[slot])

    def o_copy(ci, slot):
        r0 = row0 + ci * C
        return pltpu.make_async_copy(
            xbuf[slot], out_hbm.at[pl.ds(r0, C)], semo[slot])

    def start_load(ci, slot):
        x_copy(ci, slot).start()
        p_copy(ci, slot).start()

    def compute(slot):
        xb, pb = xbuf[slot], pbuf[slot]

        @plsc.parallel_loop(0, C * G, unroll=4)
        def _i(i):
            r = i & (C - 1)
            g = i >> LOG2C
            col = g * LANES
            p = pb[r, pl.ds(col, LANES)]
            for b in range(B):
                xb[r, b, pl.ds(col, LANES)] = (
                    xb[r, b, pl.ds(col, LANES)] + p)

    def process(ci, slot):
        @pl.when(ci >= 2)
        def _():
            o_copy(ci - 2, (slot - 2) % NSLOT).wait()

        @pl.when(ci + 2 < NCHUNK)
        def _():
            start_load(ci + 2, (slot + 2) % NSLOT)

        x_copy(ci, slot).wait()
        p_copy(ci, slot).wait()
        compute(slot)
        o_copy(ci, slot).start()

    start_load(0, 0)
    start_load(1, 1)

    def ring_body(qi, carry):
        ci = qi * NSLOT
        for s in range(NSLOT):
            process(ci + s, s)
        return carry

    lax.fori_loop(0, NCHUNK // NSLOT, ring_body, 0)

    o_copy(NCHUNK - 2, (NCHUNK - 2) % NSLOT).wait()
    o_copy(NCHUNK - 1, (NCHUNK - 1) % NSLOT).wait()


def kernel(x, pos_table):
    return _pos_add(x, pos_table)


# speedup vs baseline: 1.3198x; 1.3198x over previous
import functools

import jax
import jax.numpy as jnp
from jax import lax
from jax.experimental import pallas as pl
from jax.experimental.pallas import tpu as pltpu
from jax.experimental.pallas import tpu_sc as plsc

T = 8192
B = 4
D = 768
NC = 2
NS = 16
NW = NC * NS
ROWS_PER_W = T // NW
C = 8
LOG2C = C.bit_length() - 1
NCHUNK = ROWS_PER_W // C
NSLOT = 4
LANES = 16
G = D // LANES

_mesh = plsc.VectorSubcoreMesh(core_axis_name="c", subcore_axis_name="s")


@functools.partial(
    pl.kernel,
    mesh=_mesh,
    out_type=jax.ShapeDtypeStruct((T, B, D), jnp.float32),
    scratch_types=(
        [pltpu.VMEM((C, B, D), jnp.float32) for _ in range(NSLOT)]
        + [pltpu.VMEM((C, D), jnp.float32) for _ in range(NSLOT)]
        + [pltpu.SemaphoreType.DMA for _ in range(3 * NSLOT)]
    ),
)
def _pos_add(x_hbm, pos_hbm, out_hbm, *scratch):
    xbuf = scratch[0:NSLOT]
    pbuf = scratch[NSLOT:2 * NSLOT]
    semx = scratch[2 * NSLOT:3 * NSLOT]
    semp = scratch[3 * NSLOT:4 * NSLOT]
    semo = scratch[4 * NSLOT:5 * NSLOT]

    wid = lax.axis_index("s") * NC + lax.axis_index("c")
    row0 = wid * ROWS_PER_W

    def x_copy(ci, slot):
        r0 = row0 + ci * C
        return pltpu.make_async_copy(
            x_hbm.at[pl.ds(r0, C)], xbuf[slot], semx[slot])

    def p_copy(ci, slot):
        r0 = row0 + ci * C
        return pltpu.make_async_copy(
            pos_hbm.at[pl.ds(r0, C)], pbuf[slot], semp[slot])

    def o_copy(ci, slot):
        r0 = row0 + ci * C
        return pltpu.make_async_copy(
            xbuf[slot], out_hbm.at[pl.ds(r0, C)], semo[slot])

    def start_load(ci, slot):
        x_copy(ci, slot).start()
        p_copy(ci, slot).start()

    def compute(slot):
        xb, pb = xbuf[slot], pbuf[slot]

        @plsc.parallel_loop(0, C * G, unroll=4)
        def _i(i):
            r = i & (C - 1)
            g = i >> LOG2C
            col = g * LANES
            p = pb[r, pl.ds(col, LANES)]
            for b in range(B):
                xb[r, b, pl.ds(col, LANES)] = (
                    xb[r, b, pl.ds(col, LANES)] + p)

    def process(ci, slot):
        @pl.when(ci >= 2)
        def _():
            o_copy(ci - 2, (slot - 2) % NSLOT).wait()

        @pl.when(ci + 2 < NCHUNK)
        def _():
            start_load(ci + 2, (slot + 2) % NSLOT)

        x_copy(ci, slot).wait()
        p_copy(ci, slot).wait()
        compute(slot)
        o_copy(ci, slot).start()

    start_load(0, 0)
    start_load(1, 1)

    def ring_body(qi, carry):
        ci = qi * NSLOT
        for s in range(NSLOT):
            process(ci + s, s)
        return carry

    lax.fori_loop(0, NCHUNK // NSLOT, ring_body, 0)

    o_copy(NCHUNK - 2, (NCHUNK - 2) % NSLOT).wait()
    o_copy(NCHUNK - 1, (NCHUNK - 1) % NSLOT).wait()


TB = 256


def _tc_body(x_ref, p_ref, o_ref):
    o_ref[...] = x_ref[...] + p_ref[...][:, None, :]


def _pos_add_tc(x, pos_table, rows):
    return pl.pallas_call(
        _tc_body,
        grid=(rows // TB,),
        in_specs=[
            pl.BlockSpec((TB, B, D), lambda i: (i, 0, 0)),
            pl.BlockSpec((TB, D), lambda i: (i, 0)),
        ],
        out_specs=pl.BlockSpec((TB, B, D), lambda i: (i, 0, 0)),
        out_shape=jax.ShapeDtypeStruct((rows, B, D), jnp.float32),
    )(x, pos_table)


def kernel(x, pos_table):
    return _pos_add_tc(x, pos_table, T)
